# baseline (speedup 1.0000x reference)
import jax
import jax.numpy as jnp
from jax import lax
from jax.experimental import pallas as pl
from jax.experimental.pallas import tpu as pltpu
from jax.experimental.pallas import tpu_sc as plsc

N_NODES = 50000
D = 512
E = 512
B = 10000
S = 10

NC, NS, L = 2, 16, 16
NW = NC * NS
B_PAD = 10240
K = 4
KS = K * S
NB = 4
PF = 3
RPW = B_PAD // NW
NCHUNK = RPW // K
NP = NCHUNK // 2
K2 = 2 * K


def _sc_gather_body(nodes_hbm, neigh_hbm, feat_hbm,
                    self_out, mean_out, *scr):
    wid = lax.axis_index("s") * NC + lax.axis_index("c")
    base = wid * RPW
    sidx, nidx = scr[0], scr[1]
    sbigs = scr[2:4]
    nbufs = scr[4:4 + NB]
    accs = scr[4 + NB:6 + NB]
    sem_gs = scr[6 + NB:8 + NB]
    sem_gn = scr[8 + NB:8 + 2 * NB]
    sem_w = scr[8 + 2 * NB:10 + 2 * NB]
    sem_ws = scr[10 + 2 * NB:12 + 2 * NB]

    pltpu.sync_copy(nodes_hbm.at[wid], sidx)
    pltpu.sync_copy(neigh_hbm.at[wid], nidx)

    inv_s = jnp.float32(1.0 / S)

    def issue_self(p, sp):
        pltpu.async_copy(feat_hbm.at[sidx.at[p]], sbigs[sp], sem_gs[sp])

    def issue_neigh(ch, q):
        pltpu.async_copy(feat_hbm.at[nidx.at[ch]], nbufs[q], sem_gn[q])

    issue_self(0, 0)
    issue_self(1, 1)
    for ch in range(PF):
        issue_neigh(ch, ch)

    def group_body(g, _):
        for pp in range(2):
            p = g * 2 + pp
            sp = pp
            prow0 = base + p * K2
            sbig, acc = sbigs[sp], accs[sp]
            pltpu.make_async_copy(feat_hbm.at[sidx.at[p]], sbig,
                                  sem_gs[sp]).wait()
            pltpu.async_copy(sbig, self_out.at[pl.ds(prow0, K2)],
                             sem_ws[sp])
            @pl.when(p >= 2)
            def _():
                pltpu.make_async_copy(
                    acc, mean_out.at[pl.ds(prow0, K2)], sem_w[sp]).wait()

            for par2 in range(2):
                ch = p * 2 + par2
                q = (pp * 2 + par2) % NB
                nbuf = nbufs[q]
                pltpu.make_async_copy(feat_hbm.at[nidx.at[ch]], nbuf,
                                      sem_gn[q]).wait()

                def row_body(r, _):
                    def j_body(j, acc_vecs):
                        rowb = r * S + j
                        return tuple(
                            acc_vecs[c] + nbuf[rowb, pl.ds(c * L, L)]
                            for c in range(D // L))
                    init = tuple(nbuf[r * S, pl.ds(c * L, L)]
                                 for c in range(D // L))
                    sums = lax.fori_loop(1, S, j_body, init)
                    for c in range(D // L):
                        acc[par2 * K + r, pl.ds(c * L, L)] = sums[c] * inv_s
                    return 0
                lax.fori_loop(0, K, row_body, 0)

                @pl.when(ch + PF < NCHUNK)
                def _():
                    issue_neigh(ch + PF, (q + PF) % NB)

            pltpu.async_copy(acc, mean_out.at[pl.ds(prow0, K2)], sem_w[sp])

            @pl.when(p + 2 < NP)
            def _():
                pltpu.make_async_copy(sbig, self_out.at[pl.ds(prow0, K2)],
                                      sem_ws[sp]).wait()
                issue_self(p + 2, sp)
        return 0

    lax.fori_loop(0, NP // 2, group_body, 0)

    for sp in range(2):
        pltpu.make_async_copy(accs[sp], mean_out.at[pl.ds(0, K2)],
                              sem_w[sp]).wait()
        pltpu.make_async_copy(sbigs[sp], self_out.at[pl.ds(0, K2)],
                              sem_ws[sp]).wait()


def _sc_gather(nodes_r, neigh_r, features):
    mesh = plsc.VectorSubcoreMesh(core_axis_name="c", subcore_axis_name="s")
    f = pl.kernel(
        _sc_gather_body,
        out_type=(
            jax.ShapeDtypeStruct((B_PAD, D), jnp.float32),
            jax.ShapeDtypeStruct((B_PAD, D), jnp.float32),
        ),
        mesh=mesh,
        scratch_types=[
            pltpu.VMEM((NP, K2), jnp.int32),
            pltpu.VMEM((NCHUNK, KS), jnp.int32),
        ] + [pltpu.VMEM((K2, D), jnp.float32)] * 2
          + [pltpu.VMEM((KS, D), jnp.float32)] * NB
          + [pltpu.VMEM((K2, D), jnp.float32)] * 2
          + [pltpu.SemaphoreType.DMA] * (6 + NB),
    )
    return f(nodes_r, neigh_r, features)


def _mm_body(ws_ref, wn_ref, self_ref, mean_ref, out_ref):
    a = lax.dot_general(ws_ref[...], self_ref[...],
                        (((1,), (1,)), ((), ())),
                        preferred_element_type=jnp.float32)
    b = lax.dot_general(wn_ref[...], mean_ref[...],
                        (((1,), (1,)), ((), ())),
                        preferred_element_type=jnp.float32)
    out_ref[...] = jnp.maximum(a + b, 0.0)


BT = 2048


def _tc_matmul(ws, wn, self_f, mean_f):
    grid = (B_PAD // BT,)
    return pl.pallas_call(
        _mm_body,
        grid=grid,
        in_specs=[
            pl.BlockSpec((E, D), lambda i: (0, 0)),
            pl.BlockSpec((E, D), lambda i: (0, 0)),
            pl.BlockSpec((BT, D), lambda i: (i, 0)),
            pl.BlockSpec((BT, D), lambda i: (i, 0)),
        ],
        out_specs=pl.BlockSpec((E, BT), lambda i: (0, i)),
        out_shape=jax.ShapeDtypeStruct((E, B), jnp.float32),
        compiler_params=pltpu.CompilerParams(
            dimension_semantics=("parallel",)),
    )(ws, wn, self_f, mean_f)


def kernel(nodes, neigh_idx, features, weight):
    nodes = nodes.astype(jnp.int32)
    neigh_idx = neigh_idx.astype(jnp.int32)
    pad_n = B_PAD - B
    pad_rows = (jnp.arange(pad_n, dtype=jnp.int32) * 37) % N_NODES
    nodes_r = jnp.concatenate([nodes, pad_rows]).reshape(NW, NP, K2)
    pad_rows2 = (jnp.arange(pad_n * S, dtype=jnp.int32) * 37) % N_NODES
    neigh_r = jnp.concatenate([neigh_idx.reshape(-1), pad_rows2]).reshape(
        NW, NCHUNK, KS)
    self_f, mean_f = _sc_gather(nodes_r, neigh_r, features)
    ws = weight[:, :D]
    wn = weight[:, D:]
    return _tc_matmul(ws, wn, self_f, mean_f)


# speedup vs baseline: 1.0127x; 1.0127x over previous
import jax
import jax.numpy as jnp
from jax import lax
from jax.experimental import pallas as pl
from jax.experimental.pallas import tpu as pltpu
from jax.experimental.pallas import tpu_sc as plsc

N_NODES = 50000
D = 512
E = 512
B = 10000
S = 10

NC, NS, L = 2, 16, 16
NW = NC * NS
B_PAD = 10240
K = 4
KS = K * S
NB = 4
PF = 3
RPW = B_PAD // NW
NCHUNK = RPW // K
NP = NCHUNK // 2
K2 = 2 * K


def _sc_gather_body(nodes_hbm, neigh_hbm, feat_hbm,
                    self_out, mean_out, *scr):
    wid = lax.axis_index("s") * NC + lax.axis_index("c")
    base = wid * RPW
    sidx, nidx = scr[0], scr[1]
    sbigs = scr[2:4]
    nbufs = scr[4:4 + NB]
    accs = scr[4 + NB:6 + NB]
    sem_gs = scr[6 + NB:8 + NB]
    sem_gn = scr[8 + NB:8 + 2 * NB]
    sem_w = scr[8 + 2 * NB:10 + 2 * NB]
    sem_ws = scr[10 + 2 * NB:12 + 2 * NB]

    pltpu.sync_copy(nodes_hbm.at[wid], sidx)
    pltpu.sync_copy(neigh_hbm.at[wid], nidx)

    inv_s = jnp.float32(1.0 / S)

    def issue_self(p, sp):
        pltpu.async_copy(feat_hbm.at[sidx.at[p]], sbigs[sp], sem_gs[sp])

    def issue_neigh(ch, q):
        pltpu.async_copy(feat_hbm.at[nidx.at[ch]], nbufs[q], sem_gn[q])

    issue_self(0, 0)
    issue_self(1, 1)
    for ch in range(PF):
        issue_neigh(ch, ch)

    def group_body(g, _):
        for pp in range(2):
            p = g * 2 + pp
            sp = pp
            prow0 = base + p * K2
            sbig, acc = sbigs[sp], accs[sp]
            pltpu.make_async_copy(feat_hbm.at[sidx.at[p]], sbig,
                                  sem_gs[sp]).wait()
            pltpu.async_copy(sbig, self_out.at[pl.ds(prow0, K2)],
                             sem_ws[sp])
            @pl.when(p >= 2)
            def _():
                pltpu.make_async_copy(
                    acc, mean_out.at[pl.ds(prow0, K2)], sem_w[sp]).wait()

            for par2 in range(2):
                ch = p * 2 + par2
                q = (pp * 2 + par2) % NB
                nbuf = nbufs[q]
                pltpu.make_async_copy(feat_hbm.at[nidx.at[ch]], nbuf,
                                      sem_gn[q]).wait()

                def row_body(r, _):
                    def j_body(j, acc_vecs):
                        rowb = r * S + j
                        return tuple(
                            acc_vecs[c] + nbuf[rowb, pl.ds(c * L, L)]
                            for c in range(D // L))
                    init = tuple(nbuf[r * S, pl.ds(c * L, L)]
                                 for c in range(D // L))
                    sums = lax.fori_loop(1, S, j_body, init)
                    for c in range(D // L):
                        acc[par2 * K + r, pl.ds(c * L, L)] = sums[c] * inv_s
                    return 0
                lax.fori_loop(0, K, row_body, 0)

                @pl.when(ch + PF < NCHUNK)
                def _():
                    issue_neigh(ch + PF, (q + PF) % NB)

            pltpu.async_copy(acc, mean_out.at[pl.ds(prow0, K2)], sem_w[sp])

            @pl.when(p + 2 < NP)
            def _():
                pltpu.make_async_copy(sbig, self_out.at[pl.ds(prow0, K2)],
                                      sem_ws[sp]).wait()
                issue_self(p + 2, sp)
        return 0

    lax.fori_loop(0, NP // 2, group_body, 0)

    for sp in range(2):
        pltpu.make_async_copy(accs[sp], mean_out.at[pl.ds(0, K2)],
                              sem_w[sp]).wait()
        pltpu.make_async_copy(sbigs[sp], self_out.at[pl.ds(0, K2)],
                              sem_ws[sp]).wait()


def _sc_gather(nodes_r, neigh_r, features):
    mesh = plsc.VectorSubcoreMesh(core_axis_name="c", subcore_axis_name="s")
    f = pl.kernel(
        _sc_gather_body,
        out_type=(
            jax.ShapeDtypeStruct((B_PAD, D), jnp.float32),
            jax.ShapeDtypeStruct((B_PAD, D), jnp.float32),
        ),
        mesh=mesh,
        scratch_types=[
            pltpu.VMEM((NP, K2), jnp.int32),
            pltpu.VMEM((NCHUNK, KS), jnp.int32),
        ] + [pltpu.VMEM((K2, D), jnp.float32)] * 2
          + [pltpu.VMEM((KS, D), jnp.float32)] * NB
          + [pltpu.VMEM((K2, D), jnp.float32)] * 2
          + [pltpu.SemaphoreType.DMA] * (6 + NB),
    )
    return f(nodes_r, neigh_r, features)


def _mm_body(ws_ref, wn_ref, self_ref, mean_ref, out_ref):
    a = lax.dot_general(ws_ref[...], self_ref[...],
                        (((1,), (1,)), ((), ())),
                        preferred_element_type=jnp.float32)
    b = lax.dot_general(wn_ref[...], mean_ref[...],
                        (((1,), (1,)), ((), ())),
                        preferred_element_type=jnp.float32)
    out_ref[...] = jnp.maximum(a + b, 0.0)


BT = 1024


def _tc_matmul(ws, wn, self_f, mean_f):
    grid = (B_PAD // BT,)
    return pl.pallas_call(
        _mm_body,
        grid=grid,
        in_specs=[
            pl.BlockSpec((E, D), lambda i: (0, 0)),
            pl.BlockSpec((E, D), lambda i: (0, 0)),
            pl.BlockSpec((BT, D), lambda i: (i, 0)),
            pl.BlockSpec((BT, D), lambda i: (i, 0)),
        ],
        out_specs=pl.BlockSpec((E, BT), lambda i: (0, i)),
        out_shape=jax.ShapeDtypeStruct((E, B), jnp.float32),
        compiler_params=pltpu.CompilerParams(
            dimension_semantics=("parallel",)),
    )(ws, wn, self_f, mean_f)


def kernel(nodes, neigh_idx, features, weight):
    nodes = nodes.astype(jnp.int32)
    neigh_idx = neigh_idx.astype(jnp.int32)
    pad_n = B_PAD - B
    pad_rows = (jnp.arange(pad_n, dtype=jnp.int32) * 37) % N_NODES
    nodes_r = jnp.concatenate([nodes, pad_rows]).reshape(NW, NP, K2)
    pad_rows2 = (jnp.arange(pad_n * S, dtype=jnp.int32) * 37) % N_NODES
    neigh_r = jnp.concatenate([neigh_idx.reshape(-1), pad_rows2]).reshape(
        NW, NCHUNK, KS)
    self_f, mean_f = _sc_gather(nodes_r, neigh_r, features)
    ws = weight[:, :D]
    wn = weight[:, D:]
    return _tc_matmul(ws, wn, self_f, mean_f)
